# baseline (speedup 1.0000x reference)
import numpy as np

import jax
import jax.numpy as jnp
from jax import lax
from jax.experimental import pallas as pl
from jax.experimental.pallas import tpu as pltpu

_BT = 128


def _shift_basis(k, out_size, in_size):
    e = np.zeros((k, out_size, in_size), np.float32)
    for d in range(k):
        for o in range(out_size):
            e[d, o, o + d] = 1.0
    return jnp.asarray(e)


def _lenet_kernel(x_ref, w1b_ref, b1b_ref, w2b_ref, b2b_ref,
                  f1w_ref, f1b_ref, f2w_ref, f2b_ref, f3w_ref, f3b_ref,
                  out_ref, a1_ref, p1_ref, a2_ref, p2_ref):
    B = x_ref.shape[0]

    w1b = w1b_ref[...]
    b1b = b1b_ref[...]
    for g in range(6):
        win = x_ref[:, pl.ds(128 * g, 256)]
        part = lax.dot_general(w1b, win, (((1,), (1,)), ((), ())),
                               preferred_element_type=jnp.float32) + b1b
        a1_ref[4 * g:4 * g + 4] = part.reshape(4, 144, B)

    m = jnp.maximum(
        jnp.maximum(a1_ref[pl.ds(0, 12, stride=2), pl.ds(0, 72, stride=2), :],
                    a1_ref[pl.ds(0, 12, stride=2), pl.ds(1, 72, stride=2), :]),
        jnp.maximum(a1_ref[pl.ds(1, 12, stride=2), pl.ds(0, 72, stride=2), :],
                    a1_ref[pl.ds(1, 12, stride=2), pl.ds(1, 72, stride=2), :]))
    p1_ref[...] = jnp.maximum(m, 0.0)

    w2b = w2b_ref[...]
    b2b = b2b_ref[...]
    for y in range(10):
        part = jnp.dot(w2b, p1_ref[y:y + 3].reshape(216, B),
                       preferred_element_type=jnp.float32) + b2b
        a2_ref[y] = part.reshape(10, 16, B)

    m2 = jnp.maximum(
        jnp.maximum(a2_ref[pl.ds(0, 5, stride=2), pl.ds(0, 5, stride=2), :, :],
                    a2_ref[pl.ds(0, 5, stride=2), pl.ds(1, 5, stride=2), :, :]),
        jnp.maximum(a2_ref[pl.ds(1, 5, stride=2), pl.ds(0, 5, stride=2), :, :],
                    a2_ref[pl.ds(1, 5, stride=2), pl.ds(1, 5, stride=2), :, :]))
    p2_ref[...] = jnp.maximum(m2, 0.0)

    h = jnp.dot(f1w_ref[...], p2_ref[...].reshape(400, B),
                preferred_element_type=jnp.float32) + f1b_ref[...]
    h = jnp.maximum(h, 0.0)
    h = jnp.dot(f2w_ref[...], h,
                preferred_element_type=jnp.float32) + f2b_ref[...]
    h = jnp.maximum(h, 0.0)
    out_ref[...] = (jnp.dot(f3w_ref[...], h,
                            preferred_element_type=jnp.float32)
                    + f3b_ref[...])


def kernel(conv1_w, conv1_b, conv2_w, conv2_b, fc1_w, fc1_b,
           fc2_w, fc2_b, fc3_w, fc3_b, x_nchw):
    B = x_nchw.shape[0]
    Bp = ((B + _BT - 1) // _BT) * _BT

    x2 = jnp.pad(x_nchw.astype(jnp.float32).reshape(B, 7, 112),
                 ((0, Bp - B), (0, 0), (0, 16))).reshape(Bp, 896)

    e1y = _shift_basis(5, 4, 8)
    e1x = _shift_basis(5, 24, 28)
    w1 = conv1_w.astype(jnp.float32).reshape(6, 5, 5)
    t1 = jnp.einsum('oab,ayi->obyi', w1, e1y)
    w1blk = jnp.einsum('obyi,bxj->yoxij', t1, e1x).reshape(576, 224)
    w1blk = jnp.pad(w1blk.reshape(576, 2, 112),
                    ((0, 0), (0, 0), (0, 16))).reshape(576, 256)
    b1blk = jnp.broadcast_to(conv1_b.astype(jnp.float32)[None, :, None],
                             (4, 6, 24)).reshape(576, 1)

    e2x = _shift_basis(3, 10, 12)
    w2 = conv2_w.astype(jnp.float32)
    w2blk = jnp.einsum('ocib,bxj->xoicj', w2, e2x).reshape(160, 216)
    b2blk = jnp.broadcast_to(conv2_b.astype(jnp.float32)[None, :],
                             (10, 16)).reshape(160, 1)

    w1p = jnp.transpose(fc1_w.astype(jnp.float32).reshape(120, 16, 5, 5),
                        (0, 2, 3, 1)).reshape(120, 400)
    b1 = fc1_b.astype(jnp.float32)[:, None]
    w2f = fc2_w.astype(jnp.float32)
    b2 = fc2_b.astype(jnp.float32)[:, None]
    w3f = fc3_w.astype(jnp.float32)
    b3 = fc3_b.astype(jnp.float32)[:, None]

    flops_per_img = 2 * (6 * 576 * 224 + 10 * 160 * 216 + 120 * 400
                         + 84 * 120 + 10 * 84)
    cost = pl.CostEstimate(
        flops=int(flops_per_img * Bp),
        transcendentals=0,
        bytes_accessed=int(4 * Bp * (784 + 10)
                           + 4 * (576 * 224 + 160 * 216 + 120 * 400
                                  + 84 * 120 + 10 * 84)))

    outT = pl.pallas_call(
        _lenet_kernel,
        out_shape=jax.ShapeDtypeStruct((10, Bp), jnp.float32),
        grid=(Bp // _BT,),
        in_specs=[
            pl.BlockSpec((_BT, 896), lambda i: (i, 0)),
            pl.BlockSpec((576, 256), lambda i: (0, 0)),
            pl.BlockSpec((576, 1), lambda i: (0, 0)),
            pl.BlockSpec((160, 216), lambda i: (0, 0)),
            pl.BlockSpec((160, 1), lambda i: (0, 0)),
            pl.BlockSpec((120, 400), lambda i: (0, 0)),
            pl.BlockSpec((120, 1), lambda i: (0, 0)),
            pl.BlockSpec((84, 120), lambda i: (0, 0)),
            pl.BlockSpec((84, 1), lambda i: (0, 0)),
            pl.BlockSpec((10, 84), lambda i: (0, 0)),
            pl.BlockSpec((10, 1), lambda i: (0, 0)),
        ],
        out_specs=pl.BlockSpec((10, _BT), lambda i: (0, i)),
        scratch_shapes=[
            pltpu.VMEM((24, 144, _BT), jnp.float32),
            pltpu.VMEM((12, 72, _BT), jnp.float32),
            pltpu.VMEM((10, 10, 16, _BT), jnp.float32),
            pltpu.VMEM((5, 5, 16, _BT), jnp.float32),
        ],
        compiler_params=pltpu.CompilerParams(
            dimension_semantics=("parallel",)),
        cost_estimate=cost,
    )(x2, w1blk, b1blk, w2blk, b2blk, w1p, b1, w2f, b2, w3f, b3)

    return outT[:, :B].T


# speedup vs baseline: 1.6445x; 1.6445x over previous
import numpy as np

import jax
import jax.numpy as jnp
from jax import lax
from jax.experimental import pallas as pl
from jax.experimental.pallas import tpu as pltpu

_BT = 128


def _shift_basis(k, out_size, in_size):
    e = np.zeros((k, out_size, in_size), np.float32)
    for d in range(k):
        for o in range(out_size):
            e[d, o, o + d] = 1.0
    return jnp.asarray(e)


def _lenet_kernel(x_ref, w1b_ref, b1b_ref, w2b_ref, b2b_ref,
                  f1w_ref, f1b_ref, f2w_ref, f2b_ref, f3w_ref, f3b_ref,
                  out_ref, a1_ref, p1_ref, a2_ref, p2_ref):
    B = x_ref.shape[-1]

    w1b = w1b_ref[...]
    b1b = b1b_ref[...]
    for g in range(6):
        win = x_ref[4 * g:4 * g + 8].reshape(256, B)
        part = jnp.dot(w1b, win,
                       preferred_element_type=jnp.float32) + b1b
        a1_ref[4 * g:4 * g + 4] = part.reshape(4, 144, B)

    m = jnp.maximum(
        jnp.maximum(a1_ref[pl.ds(0, 12, stride=2), pl.ds(0, 72, stride=2), :],
                    a1_ref[pl.ds(0, 12, stride=2), pl.ds(1, 72, stride=2), :]),
        jnp.maximum(a1_ref[pl.ds(1, 12, stride=2), pl.ds(0, 72, stride=2), :],
                    a1_ref[pl.ds(1, 12, stride=2), pl.ds(1, 72, stride=2), :]))
    p1_ref[...] = jnp.maximum(m, 0.0)

    w2b = w2b_ref[...]
    b2b = b2b_ref[...]
    for y in range(10):
        part = jnp.dot(w2b, p1_ref[y:y + 3].reshape(216, B),
                       preferred_element_type=jnp.float32) + b2b
        a2_ref[y] = part.reshape(10, 16, B)

    m2 = jnp.maximum(
        jnp.maximum(a2_ref[pl.ds(0, 5, stride=2), pl.ds(0, 5, stride=2), :, :],
                    a2_ref[pl.ds(0, 5, stride=2), pl.ds(1, 5, stride=2), :, :]),
        jnp.maximum(a2_ref[pl.ds(1, 5, stride=2), pl.ds(0, 5, stride=2), :, :],
                    a2_ref[pl.ds(1, 5, stride=2), pl.ds(1, 5, stride=2), :, :]))
    p2_ref[...] = jnp.maximum(m2, 0.0)

    h = jnp.dot(f1w_ref[...], p2_ref[...].reshape(400, B),
                preferred_element_type=jnp.float32) + f1b_ref[...]
    h = jnp.maximum(h, 0.0)
    h = jnp.dot(f2w_ref[...], h,
                preferred_element_type=jnp.float32) + f2b_ref[...]
    h = jnp.maximum(h, 0.0)
    out_ref[...] = (jnp.dot(f3w_ref[...], h,
                            preferred_element_type=jnp.float32)
                    + f3b_ref[...])


def kernel(conv1_w, conv1_b, conv2_w, conv2_b, fc1_w, fc1_b,
           fc2_w, fc2_b, fc3_w, fc3_b, x_nchw):
    B = x_nchw.shape[0]
    Bp = ((B + _BT - 1) // _BT) * _BT

    xT = jnp.transpose(x_nchw[:, 0, :, :].astype(jnp.float32), (1, 2, 0))
    xT = jnp.pad(xT, ((0, 0), (0, 4), (0, Bp - B)))

    e1y = _shift_basis(5, 4, 8)
    e1x = _shift_basis(5, 24, 28)
    w1 = conv1_w.astype(jnp.float32).reshape(6, 5, 5)
    t1 = jnp.einsum('oab,ayi->obyi', w1, e1y)
    w1blk = jnp.einsum('obyi,bxj->yoxij', t1, e1x).reshape(576, 224)
    w1blk = jnp.pad(w1blk.reshape(576, 8, 28),
                    ((0, 0), (0, 0), (0, 4))).reshape(576, 256)
    b1blk = jnp.broadcast_to(conv1_b.astype(jnp.float32)[None, :, None],
                             (4, 6, 24)).reshape(576, 1)

    e2x = _shift_basis(3, 10, 12)
    w2 = conv2_w.astype(jnp.float32)
    w2blk = jnp.einsum('ocib,bxj->xoicj', w2, e2x).reshape(160, 216)
    b2blk = jnp.broadcast_to(conv2_b.astype(jnp.float32)[None, :],
                             (10, 16)).reshape(160, 1)

    w1p = jnp.transpose(fc1_w.astype(jnp.float32).reshape(120, 16, 5, 5),
                        (0, 2, 3, 1)).reshape(120, 400)
    b1 = fc1_b.astype(jnp.float32)[:, None]
    w2f = fc2_w.astype(jnp.float32)
    b2 = fc2_b.astype(jnp.float32)[:, None]
    w3f = fc3_w.astype(jnp.float32)
    b3 = fc3_b.astype(jnp.float32)[:, None]

    flops_per_img = 2 * (6 * 576 * 224 + 10 * 160 * 216 + 120 * 400
                         + 84 * 120 + 10 * 84)
    cost = pl.CostEstimate(
        flops=int(flops_per_img * Bp),
        transcendentals=0,
        bytes_accessed=int(4 * Bp * (784 + 10)
                           + 4 * (576 * 224 + 160 * 216 + 120 * 400
                                  + 84 * 120 + 10 * 84)))

    outT = pl.pallas_call(
        _lenet_kernel,
        out_shape=jax.ShapeDtypeStruct((10, Bp), jnp.float32),
        grid=(Bp // _BT,),
        in_specs=[
            pl.BlockSpec((28, 32, _BT), lambda i: (0, 0, i)),
            pl.BlockSpec((576, 256), lambda i: (0, 0)),
            pl.BlockSpec((576, 1), lambda i: (0, 0)),
            pl.BlockSpec((160, 216), lambda i: (0, 0)),
            pl.BlockSpec((160, 1), lambda i: (0, 0)),
            pl.BlockSpec((120, 400), lambda i: (0, 0)),
            pl.BlockSpec((120, 1), lambda i: (0, 0)),
            pl.BlockSpec((84, 120), lambda i: (0, 0)),
            pl.BlockSpec((84, 1), lambda i: (0, 0)),
            pl.BlockSpec((10, 84), lambda i: (0, 0)),
            pl.BlockSpec((10, 1), lambda i: (0, 0)),
        ],
        out_specs=pl.BlockSpec((10, _BT), lambda i: (0, i)),
        scratch_shapes=[
            pltpu.VMEM((24, 144, _BT), jnp.float32),
            pltpu.VMEM((12, 72, _BT), jnp.float32),
            pltpu.VMEM((10, 10, 16, _BT), jnp.float32),
            pltpu.VMEM((5, 5, 16, _BT), jnp.float32),
        ],
        compiler_params=pltpu.CompilerParams(
            dimension_semantics=("parallel",)),
        cost_estimate=cost,
    )(xT, w1blk, b1blk, w2blk, b2blk, w1p, b1, w2f, b2, w3f, b3)

    return outT[:, :B].T
